# baseline (speedup 1.0000x reference)
import functools

import jax
import jax.numpy as jnp
from jax import lax
from jax.experimental import pallas as pl
from jax.experimental.pallas import tpu as pltpu
from jax.experimental.pallas import tpu_sc as plsc

EPS = 1e-12

NC = 2
NS = 16
NW = NC * NS


def _sc_gather(table, ids_flat, chunk=16):
    n = ids_flat.shape[0]
    d = table.shape[1]
    b_per_w = n // NW
    nchunks = b_per_w // chunk
    assert nchunks % 2 == 0 and nchunks >= 4
    mesh = plsc.VectorSubcoreMesh(core_axis_name="c", subcore_axis_name="s")

    nbuf = 4
    assert nchunks >= nbuf and nchunks % nbuf == 0

    @functools.partial(
        pl.kernel,
        mesh=mesh,
        out_type=jax.ShapeDtypeStruct((n, d), jnp.float32),
        scratch_types=(
            [pltpu.VMEM((chunk,), jnp.int32) for _ in range(nbuf)]
            + [pltpu.VMEM((chunk, d), jnp.float32) for _ in range(nbuf)]
            + [pltpu.SemaphoreType.DMA for _ in range(2 * nbuf)]
        ),
    )
    def k(table_hbm, idx_hbm, out_hbm, *scr):
        idx_v = scr[0:nbuf]
        rows_v = scr[nbuf:2 * nbuf]
        gsem = scr[2 * nbuf:3 * nbuf]
        ssem = scr[3 * nbuf:4 * nbuf]
        wid = lax.axis_index("s") * NC + lax.axis_index("c")
        base = wid * b_per_w

        def start_gather(c, b):
            pltpu.sync_copy(idx_hbm.at[pl.ds(base + c * chunk, chunk)], idx_v[b])
            pltpu.make_async_copy(table_hbm.at[idx_v[b]], rows_v[b], gsem[b]).start()

        def wait_scatter(c, b):
            pltpu.make_async_copy(
                rows_v[b], out_hbm.at[pl.ds(base + c * chunk, chunk)],
                ssem[b]).wait()

        for c in range(nbuf):
            start_gather(c, c)

        @pl.loop(0, nchunks, step=nbuf)
        def _(c0):
            for b in range(nbuf):
                ci = c0 + b
                pltpu.make_async_copy(
                    table_hbm.at[idx_v[b]], rows_v[b], gsem[b]).wait()
                pltpu.make_async_copy(
                    rows_v[b], out_hbm.at[pl.ds(base + ci * chunk, chunk)],
                    ssem[b]).start()
                pb = (b - 1) % nbuf

                @pl.when(jnp.logical_and(ci >= 1, ci + nbuf - 1 < nchunks))
                def _():
                    wait_scatter(ci - 1, pb)
                    start_gather(ci + nbuf - 1, pb)

        for c in range(nchunks - nbuf, nchunks):
            wait_scatter(c, c % nbuf)

    return k(table, ids_flat)


def _tc_body(prev_ref, word_ref, tt_ref, pos_ref, type_ref, w_ref, b_ref,
             out_ref):
    del prev_ref
    ttf = tt_ref[0, 0, :][:, None]
    t0 = type_ref[0, :][None, :]
    dt = (type_ref[1, :] - type_ref[0, :])[None, :]
    x = word_ref[...] + pos_ref[...] + t0 + ttf * dt
    mu = jnp.mean(x, axis=1, keepdims=True)
    xc = x - mu
    var = jnp.mean(xc * xc, axis=1, keepdims=True)
    out_ref[...] = xc * lax.rsqrt(var + EPS) * w_ref[...] + b_ref[...]


def kernel(input_ids, token_type_ids, word_table, pos_table, type_table,
           ln_weight, ln_bias):
    B, S = input_ids.shape
    H = word_table.shape[1]
    n = B * S
    K = 8
    BK = B // K
    nk = BK * S

    ids_flat = input_ids.reshape(n).astype(jnp.int32)
    ttf = token_type_ids.astype(jnp.float32).reshape(B, 1, S)
    w2 = ln_weight.reshape(1, H)
    b2 = ln_bias.reshape(1, H)

    word_chunks = [
        _sc_gather(word_table, lax.slice_in_dim(ids_flat, k * nk, (k + 1) * nk))
        for k in range(K)
    ]

    out = None
    for k in range(K):
        off = k * BK
        body = _tc_body if k else functools.partial(_tc_body, None)
        specs = [
            pl.BlockSpec((S, H), lambda i: (i, 0)),
            pl.BlockSpec((1, 1, S), lambda i, off=off: (i + off, 0, 0)),
            pl.BlockSpec((S, H), lambda i: (0, 0)),
            pl.BlockSpec((2, H), lambda i: (0, 0)),
            pl.BlockSpec((1, H), lambda i: (0, 0)),
            pl.BlockSpec((1, H), lambda i: (0, 0)),
        ]
        args = [word_chunks[k], ttf, pos_table, type_table, w2, b2]
        if k:
            specs.insert(0, pl.BlockSpec(memory_space=pl.ANY))
            args.insert(0, out)
        out = pl.pallas_call(
            body,
            grid=(BK,),
            in_specs=specs,
            out_specs=pl.BlockSpec((S, H), lambda i, off=off: (i + off, 0)),
            out_shape=jax.ShapeDtypeStruct((n, H), jnp.float32),
            input_output_aliases={0: 0} if k else {},
        )(*args)

    return out.reshape(B, S, H)


# speedup vs baseline: 1.0677x; 1.0677x over previous
import functools

import jax
import jax.numpy as jnp
from jax import lax
from jax.experimental import pallas as pl
from jax.experimental.pallas import tpu as pltpu
from jax.experimental.pallas import tpu_sc as plsc

EPS = 1e-12

NC = 2
NS = 16
NW = NC * NS


def _sc_gather(table, ids_flat, chunk=16):
    n = ids_flat.shape[0]
    d = table.shape[1]
    b_per_w = n // NW
    nchunks = b_per_w // chunk
    assert nchunks % 2 == 0 and nchunks >= 4
    mesh = plsc.VectorSubcoreMesh(core_axis_name="c", subcore_axis_name="s")

    nbuf = 4
    assert nchunks >= nbuf and nchunks % nbuf == 0

    @functools.partial(
        pl.kernel,
        mesh=mesh,
        out_type=jax.ShapeDtypeStruct((n, d), jnp.float32),
        scratch_types=(
            [pltpu.VMEM((chunk,), jnp.int32) for _ in range(nbuf)]
            + [pltpu.VMEM((chunk, d), jnp.float32) for _ in range(nbuf)]
            + [pltpu.SemaphoreType.DMA for _ in range(2 * nbuf)]
        ),
    )
    def k(table_hbm, idx_hbm, out_hbm, *scr):
        idx_v = scr[0:nbuf]
        rows_v = scr[nbuf:2 * nbuf]
        gsem = scr[2 * nbuf:3 * nbuf]
        ssem = scr[3 * nbuf:4 * nbuf]
        wid = lax.axis_index("s") * NC + lax.axis_index("c")
        base = wid * b_per_w

        def start_gather(c, b):
            pltpu.sync_copy(idx_hbm.at[pl.ds(base + c * chunk, chunk)], idx_v[b])
            pltpu.make_async_copy(table_hbm.at[idx_v[b]], rows_v[b], gsem[b]).start()

        def wait_scatter(c, b):
            pltpu.make_async_copy(
                rows_v[b], out_hbm.at[pl.ds(base + c * chunk, chunk)],
                ssem[b]).wait()

        for c in range(nbuf):
            start_gather(c, c)

        @pl.loop(0, nchunks, step=nbuf)
        def _(c0):
            for b in range(nbuf):
                ci = c0 + b
                pltpu.make_async_copy(
                    table_hbm.at[idx_v[b]], rows_v[b], gsem[b]).wait()
                pltpu.make_async_copy(
                    rows_v[b], out_hbm.at[pl.ds(base + ci * chunk, chunk)],
                    ssem[b]).start()
                pb = (b - 1) % nbuf

                @pl.when(jnp.logical_and(ci >= 1, ci + nbuf - 1 < nchunks))
                def _():
                    wait_scatter(ci - 1, pb)
                    start_gather(ci + nbuf - 1, pb)

        for c in range(nchunks - nbuf, nchunks):
            wait_scatter(c, c % nbuf)

    return k(table, ids_flat)


def _tc_body(prev_ref, word_ref, tt_ref, pos_ref, type_ref, w_ref, b_ref,
             out_ref):
    del prev_ref
    ttf = tt_ref[0, 0, :][:, None]
    t0 = type_ref[0, :][None, :]
    dt = (type_ref[1, :] - type_ref[0, :])[None, :]
    x = word_ref[...] + pos_ref[...] + t0 + ttf * dt
    mu = jnp.mean(x, axis=1, keepdims=True)
    xc = x - mu
    var = jnp.mean(xc * xc, axis=1, keepdims=True)
    out_ref[...] = xc * lax.rsqrt(var + EPS) * w_ref[...] + b_ref[...]


def kernel(input_ids, token_type_ids, word_table, pos_table, type_table,
           ln_weight, ln_bias):
    B, S = input_ids.shape
    H = word_table.shape[1]
    n = B * S
    K = 4
    BK = B // K
    nk = BK * S

    ids_flat = input_ids.reshape(n).astype(jnp.int32)
    ttf = token_type_ids.astype(jnp.float32).reshape(B, 1, S)
    w2 = ln_weight.reshape(1, H)
    b2 = ln_bias.reshape(1, H)

    word_chunks = [
        _sc_gather(word_table, lax.slice_in_dim(ids_flat, k * nk, (k + 1) * nk))
        for k in range(K)
    ]

    out = None
    for k in range(K):
        off = k * BK
        body = _tc_body if k else functools.partial(_tc_body, None)
        specs = [
            pl.BlockSpec((S, H), lambda i: (i, 0)),
            pl.BlockSpec((1, 1, S), lambda i, off=off: (i + off, 0, 0)),
            pl.BlockSpec((S, H), lambda i: (0, 0)),
            pl.BlockSpec((2, H), lambda i: (0, 0)),
            pl.BlockSpec((1, H), lambda i: (0, 0)),
            pl.BlockSpec((1, H), lambda i: (0, 0)),
        ]
        args = [word_chunks[k], ttf, pos_table, type_table, w2, b2]
        if k:
            specs.insert(0, pl.BlockSpec(memory_space=pl.ANY))
            args.insert(0, out)
        out = pl.pallas_call(
            body,
            grid=(BK,),
            in_specs=specs,
            out_specs=pl.BlockSpec((S, H), lambda i, off=off: (i + off, 0)),
            out_shape=jax.ShapeDtypeStruct((n, H), jnp.float32),
            input_output_aliases={0: 0} if k else {},
        )(*args)

    return out.reshape(B, S, H)


# speedup vs baseline: 1.0711x; 1.0032x over previous
import functools

import jax
import jax.numpy as jnp
from jax import lax
from jax.experimental import pallas as pl
from jax.experimental.pallas import tpu as pltpu
from jax.experimental.pallas import tpu_sc as plsc

EPS = 1e-12

NC = 2
NS = 16
NW = NC * NS


def _sc_gather(table, ids_flat, n, ids_offset, chunk=16):
    d = table.shape[1]
    b_per_w = n // NW
    nchunks = b_per_w // chunk
    assert nchunks % 2 == 0 and nchunks >= 4
    mesh = plsc.VectorSubcoreMesh(core_axis_name="c", subcore_axis_name="s")

    nbuf = 4
    assert nchunks >= nbuf and nchunks % nbuf == 0

    @functools.partial(
        pl.kernel,
        mesh=mesh,
        out_type=jax.ShapeDtypeStruct((n, d), jnp.float32),
        scratch_types=(
            [pltpu.VMEM((chunk,), jnp.int32) for _ in range(nbuf)]
            + [pltpu.VMEM((chunk, d), jnp.float32) for _ in range(nbuf)]
            + [pltpu.SemaphoreType.DMA for _ in range(2 * nbuf)]
        ),
    )
    def k(table_hbm, idx_hbm, out_hbm, *scr):
        idx_v = scr[0:nbuf]
        rows_v = scr[nbuf:2 * nbuf]
        gsem = scr[2 * nbuf:3 * nbuf]
        ssem = scr[3 * nbuf:4 * nbuf]
        wid = lax.axis_index("s") * NC + lax.axis_index("c")
        base = wid * b_per_w

        def start_gather(c, b):
            pltpu.sync_copy(
                idx_hbm.at[pl.ds(ids_offset + base + c * chunk, chunk)], idx_v[b])
            pltpu.make_async_copy(table_hbm.at[idx_v[b]], rows_v[b], gsem[b]).start()

        def wait_scatter(c, b):
            pltpu.make_async_copy(
                rows_v[b], out_hbm.at[pl.ds(base + c * chunk, chunk)],
                ssem[b]).wait()

        for c in range(nbuf):
            start_gather(c, c)

        @pl.loop(0, nchunks, step=nbuf)
        def _(c0):
            for b in range(nbuf):
                ci = c0 + b
                pltpu.make_async_copy(
                    table_hbm.at[idx_v[b]], rows_v[b], gsem[b]).wait()
                pltpu.make_async_copy(
                    rows_v[b], out_hbm.at[pl.ds(base + ci * chunk, chunk)],
                    ssem[b]).start()
                pb = (b - 1) % nbuf

                @pl.when(jnp.logical_and(ci >= 1, ci + nbuf - 1 < nchunks))
                def _():
                    wait_scatter(ci - 1, pb)
                    start_gather(ci + nbuf - 1, pb)

        for c in range(nchunks - nbuf, nchunks):
            wait_scatter(c, c % nbuf)

    return k(table, ids_flat)


def _tc_body(prev_ref, word_ref, tt_ref, pos_ref, type_ref, w_ref, b_ref,
             out_ref):
    del prev_ref
    ttf = tt_ref[0, 0, :].astype(jnp.float32)[:, None]
    t0 = type_ref[0, :][None, :]
    dt = (type_ref[1, :] - type_ref[0, :])[None, :]
    x = word_ref[...] + pos_ref[...] + t0 + ttf * dt
    mu = jnp.mean(x, axis=1, keepdims=True)
    xc = x - mu
    var = jnp.mean(xc * xc, axis=1, keepdims=True)
    out_ref[...] = xc * lax.rsqrt(var + EPS) * w_ref[...] + b_ref[...]


def kernel(input_ids, token_type_ids, word_table, pos_table, type_table,
           ln_weight, ln_bias):
    B, S = input_ids.shape
    H = word_table.shape[1]
    n = B * S
    K = 4
    BK = B // K
    nk = BK * S

    ids_flat = input_ids.reshape(n).astype(jnp.int32)
    ttf = token_type_ids.astype(jnp.int32).reshape(B, 1, S)
    w2 = ln_weight.reshape(1, H)
    b2 = ln_bias.reshape(1, H)

    word_chunks = [
        _sc_gather(word_table, ids_flat, nk, k * nk) for k in range(K)
    ]

    out = None
    for k in range(K):
        off = k * BK
        body = _tc_body if k else functools.partial(_tc_body, None)
        specs = [
            pl.BlockSpec((S, H), lambda i: (i, 0)),
            pl.BlockSpec((1, 1, S), lambda i, off=off: (i + off, 0, 0)),
            pl.BlockSpec((S, H), lambda i: (0, 0)),
            pl.BlockSpec((2, H), lambda i: (0, 0)),
            pl.BlockSpec((1, H), lambda i: (0, 0)),
            pl.BlockSpec((1, H), lambda i: (0, 0)),
        ]
        args = [word_chunks[k], ttf, pos_table, type_table, w2, b2]
        if k:
            specs.insert(0, pl.BlockSpec(memory_space=pl.ANY))
            args.insert(0, out)
        out = pl.pallas_call(
            body,
            grid=(BK,),
            in_specs=specs,
            out_specs=pl.BlockSpec((S, H), lambda i, off=off: (i + off, 0)),
            out_shape=jax.ShapeDtypeStruct((n, H), jnp.float32),
            input_output_aliases={0: 0} if k else {},
        )(*args)

    return out.reshape(B, S, H)


# speedup vs baseline: 1.0725x; 1.0013x over previous
import functools

import jax
import jax.numpy as jnp
from jax import lax
from jax.experimental import pallas as pl
from jax.experimental.pallas import tpu as pltpu
from jax.experimental.pallas import tpu_sc as plsc

EPS = 1e-12

NC = 2
NS = 16
NW = NC * NS


def _sc_gather(table, ids_flat, n, ids_offset, chunk=16):
    d = table.shape[1]
    b_per_w = n // NW
    nchunks = b_per_w // chunk
    assert nchunks % 2 == 0 and nchunks >= 4
    mesh = plsc.VectorSubcoreMesh(core_axis_name="c", subcore_axis_name="s")

    nbuf = 4
    assert nchunks >= nbuf and nchunks % nbuf == 0

    @functools.partial(
        pl.kernel,
        mesh=mesh,
        out_type=jax.ShapeDtypeStruct((n, d), jnp.float32),
        scratch_types=(
            [pltpu.VMEM((chunk,), jnp.int32) for _ in range(nbuf)]
            + [pltpu.VMEM((chunk, d), jnp.float32) for _ in range(nbuf)]
            + [pltpu.SemaphoreType.DMA for _ in range(2 * nbuf)]
        ),
    )
    def k(table_hbm, idx_hbm, out_hbm, *scr):
        idx_v = scr[0:nbuf]
        rows_v = scr[nbuf:2 * nbuf]
        gsem = scr[2 * nbuf:3 * nbuf]
        ssem = scr[3 * nbuf:4 * nbuf]
        wid = lax.axis_index("s") * NC + lax.axis_index("c")
        base = wid * b_per_w

        def start_gather(c, b):
            pltpu.sync_copy(
                idx_hbm.at[pl.ds(ids_offset + base + c * chunk, chunk)], idx_v[b])
            pltpu.make_async_copy(table_hbm.at[idx_v[b]], rows_v[b], gsem[b]).start()

        def wait_scatter(c, b):
            pltpu.make_async_copy(
                rows_v[b], out_hbm.at[pl.ds(base + c * chunk, chunk)],
                ssem[b]).wait()

        for c in range(nbuf):
            start_gather(c, c)

        @pl.loop(0, nchunks, step=nbuf)
        def _(c0):
            for b in range(nbuf):
                ci = c0 + b
                pltpu.make_async_copy(
                    table_hbm.at[idx_v[b]], rows_v[b], gsem[b]).wait()
                pltpu.make_async_copy(
                    rows_v[b], out_hbm.at[pl.ds(base + ci * chunk, chunk)],
                    ssem[b]).start()
                pb = (b - 1) % nbuf

                @pl.when(jnp.logical_and(ci >= 1, ci + nbuf - 1 < nchunks))
                def _():
                    wait_scatter(ci - 1, pb)
                    start_gather(ci + nbuf - 1, pb)

        for c in range(nchunks - nbuf, nchunks):
            wait_scatter(c, c % nbuf)

    return k(table, ids_flat)


def _tc_body(prev_ref, word_ref, tt_ref, pos_ref, type_ref, w_ref, b_ref,
             out_ref):
    del prev_ref
    ttf = tt_ref[0, 0, :].astype(jnp.float32)[:, None]
    t0 = type_ref[0, :][None, :]
    dt = (type_ref[1, :] - type_ref[0, :])[None, :]
    x = word_ref[...] + pos_ref[...] + t0 + ttf * dt
    mu = jnp.mean(x, axis=1, keepdims=True)
    xc = x - mu
    var = jnp.mean(xc * xc, axis=1, keepdims=True)
    out_ref[...] = xc * lax.rsqrt(var + EPS) * w_ref[...] + b_ref[...]


def kernel(input_ids, token_type_ids, word_table, pos_table, type_table,
           ln_weight, ln_bias):
    B, S = input_ids.shape
    H = word_table.shape[1]
    n = B * S
    K = 4
    BK = B // K
    nk = BK * S

    ids_flat = input_ids.reshape(n).astype(jnp.int32)
    tt3 = token_type_ids.astype(jnp.int32).reshape(B, 1, S)
    w2 = ln_weight.reshape(1, H)
    b2 = ln_bias.reshape(1, H)

    word_chunks = [
        _sc_gather(word_table, ids_flat, nk, k * nk) for k in range(K)
    ]

    out = None
    for k in range(K):
        off = k * BK
        body = _tc_body if k else functools.partial(_tc_body, None)
        specs = [
            pl.BlockSpec((S, H), lambda i: (i, 0)),
            pl.BlockSpec((1, 1, S), lambda i, off=off: (i + off, 0, 0)),
            pl.BlockSpec((S, H), lambda i: (0, 0)),
            pl.BlockSpec((2, H), lambda i: (0, 0)),
            pl.BlockSpec((1, H), lambda i: (0, 0)),
            pl.BlockSpec((1, H), lambda i: (0, 0)),
        ]
        args = [word_chunks[k], tt3, pos_table, type_table, w2, b2]
        if k:
            specs.insert(0, pl.BlockSpec(memory_space=pl.ANY))
            args.insert(0, out)
        out = pl.pallas_call(
            body,
            grid=(BK,),
            in_specs=specs,
            out_specs=pl.BlockSpec((S, H), lambda i, off=off: (i + off, 0)),
            out_shape=jax.ShapeDtypeStruct((n, H), jnp.float32),
            input_output_aliases={0: 0} if k else {},
        )(*args)

    return out.reshape(B, S, H)
